# baseline (speedup 1.0000x reference)
import functools

import jax
import jax.numpy as jnp
from jax import lax
from jax.experimental import pallas as pl
from jax.experimental.pallas import tpu as pltpu
from jax.experimental.pallas import tpu_sc as plsc

D = 128
DW = D // 2
NC = 2
NS = 16
L = 16
NW = NC * NS
EC = 32

_mesh = plsc.VectorSubcoreMesh(
    core_axis_name="c", subcore_axis_name="s", num_cores=NC, num_subcores=NS
)


def _make_sc_agg(n_nodes: int, ch: int, r_pad: int):
    @functools.partial(
        pl.kernel,
        out_type=jax.ShapeDtypeStruct((NC, r_pad, D), jnp.float32),
        mesh=_mesh,
        scratch_types=[
            pltpu.VMEM((ch // 2, 2 * EC), jnp.int32),
            pltpu.VMEM((8, EC), jnp.int32),
            pltpu.VMEM((8, EC), jnp.int32),
            pltpu.VMEM((EC, D), jnp.float32),
            pltpu.VMEM((128, D), jnp.float32),
            pltpu.VMEM_SHARED((r_pad, D), jnp.float32),
            pltpu.SemaphoreType.DMA,
        ],
    )
    def _sc_agg(
        x_hbm, pk_hbm, out_hbm, pk_v, srcb, dstb, rows, zb, acc, sem
    ):
        c = lax.axis_index("c")
        s = lax.axis_index("s")
        wid = s * NC + c

        pltpu.sync_copy(pk_hbm.at[wid], pk_v)

        def _zstep(r, _):
            for k in range(D // L):
                zb[r, pl.ds(k * L, L)] = jnp.zeros((L,), jnp.float32)
            return ()

        lax.fori_loop(0, 128, _zstep, ())
        zrows = r_pad // NS
        r0 = s * zrows
        zfull, zrem = divmod(zrows, 128)
        for z in range(zfull):
            pltpu.sync_copy(zb, acc.at[pl.ds(r0 + z * 128, 128)])
        if zrem:
            pltpu.sync_copy(
                zb.at[pl.ds(0, zrem)], acc.at[pl.ds(r0 + zfull * 128, zrem)]
            )
        plsc.subcore_barrier()

        def _step(j2, _):
            for b in range(2):
                off = b * EC
                for k in range(EC // L):
                    wz = pk_v[j2, pl.ds(off + k * L, L)]
                    srcb[0, pl.ds(k * L, L)] = jnp.bitwise_and(wz, 0xFFFF)
                    dstb[0, pl.ds(k * L, L)] = jnp.right_shift(wz, 16)
                pltpu.async_copy(x_hbm.at[srcb.at[0]], rows, sem).wait()
                pltpu.sync_copy(rows, acc.at[dstb.at[0]], add=True)
            return ()

        lax.fori_loop(0, ch // 2, _step, ())
        plsc.subcore_barrier()

        for z in range(zfull):
            pltpu.sync_copy(acc.at[pl.ds(r0 + z * 128, 128)], zb)
            pltpu.sync_copy(zb, out_hbm.at[c, pl.ds(r0 + z * 128, 128)])
        if zrem:
            pltpu.sync_copy(
                acc.at[pl.ds(r0 + zfull * 128, zrem)], zb.at[pl.ds(0, zrem)]
            )
            pltpu.sync_copy(
                zb.at[pl.ds(0, zrem)],
                out_hbm.at[c, pl.ds(r0 + zfull * 128, zrem)],
            )

    return _sc_agg


def _tc_body(p_ref, w_ref, b_ref, o_ref):
    o_ref[...] = (
        jnp.dot(
            p_ref[0] + p_ref[1], w_ref[...], preferred_element_type=jnp.float32
        )
        + b_ref[...]
    )


def _tc_combine(partials, Wp, bias, n_nodes: int):
    bm = 2000
    return pl.pallas_call(
        _tc_body,
        grid=(n_nodes // bm,),
        in_specs=[
            pl.BlockSpec((NC, bm, D), lambda i: (0, i, 0)),
            pl.BlockSpec((D, D), lambda i: (0, 0)),
            pl.BlockSpec((1, D), lambda i: (0, 0)),
        ],
        out_specs=pl.BlockSpec((bm, D), lambda i: (i, 0)),
        out_shape=jax.ShapeDtypeStruct((n_nodes, D), jnp.float32),
    )(partials, Wp, bias.reshape(1, D))


def kernel(x, edge_index, W, bias):
    n = x.shape[0]
    e = edge_index.shape[1]
    src = edge_index[0].astype(jnp.int32)
    dst = edge_index[1].astype(jnp.int32)

    block = NW * EC
    ch = (e + block - 1) // block
    ch += ch % 2
    e_pad = block * ch
    pad = e_pad - e
    src = jnp.concatenate([src, jnp.zeros((pad,), jnp.int32)])
    dst = jnp.concatenate([dst, jnp.full((pad,), n, jnp.int32)])
    packed = jnp.bitwise_or(src, jnp.left_shift(dst, 16))
    pk3 = packed.reshape(NW, ch // 2, 2 * EC)

    r_pad = ((n + 1 + 127) // 128) * 128
    partials = _make_sc_agg(n, ch, r_pad)(x, pk3)
    return _tc_combine(partials, W, bias, n)


# speedup vs baseline: 1.2724x; 1.2724x over previous
import functools

import jax
import jax.numpy as jnp
from jax import lax
from jax.experimental import pallas as pl
from jax.experimental.pallas import tpu as pltpu
from jax.experimental.pallas import tpu_sc as plsc

D = 128
DW = D // 2
NC = 2
NS = 16
L = 16
NW = NC * NS
EC = 64

_mesh = plsc.VectorSubcoreMesh(
    core_axis_name="c", subcore_axis_name="s", num_cores=NC, num_subcores=NS
)


def _make_sc_agg(n_nodes: int, ch: int, r_pad: int):
    @functools.partial(
        pl.kernel,
        out_type=jax.ShapeDtypeStruct((NC, r_pad, D), jnp.float32),
        mesh=_mesh,
        scratch_types=[
            pltpu.VMEM((ch // 2, 2 * EC), jnp.int32),
            pltpu.VMEM((8, EC), jnp.int32),
            pltpu.VMEM((8, EC), jnp.int32),
            pltpu.VMEM((2, EC, D), jnp.float32),
            pltpu.VMEM((128, D), jnp.float32),
            pltpu.VMEM_SHARED((r_pad, D), jnp.float32),
            pltpu.SemaphoreType.DMA,
            pltpu.SemaphoreType.DMA,
        ],
    )
    def _sc_agg(
        x_hbm, pk_hbm, out_hbm, pk_v, srcb, dstb, rows, zb, acc, sem0, sem1
    ):
        c = lax.axis_index("c")
        s = lax.axis_index("s")
        wid = s * NC + c

        pltpu.sync_copy(pk_hbm.at[wid], pk_v)

        def _zstep(r, _):
            for k in range(D // L):
                zb[r, pl.ds(k * L, L)] = jnp.zeros((L,), jnp.float32)
            return ()

        lax.fori_loop(0, 128, _zstep, ())
        zrows = r_pad // NS
        r0 = s * zrows
        zfull, zrem = divmod(zrows, 128)
        for z in range(zfull):
            pltpu.sync_copy(zb, acc.at[pl.ds(r0 + z * 128, 128)])
        if zrem:
            pltpu.sync_copy(
                zb.at[pl.ds(0, zrem)], acc.at[pl.ds(r0 + zfull * 128, zrem)]
            )
        plsc.subcore_barrier()

        sems = (sem0, sem1)

        def _unpack(row, off, b):
            for k in range(EC // L):
                wz = pk_v[row, pl.ds(off + k * L, L)]
                srcb[b, pl.ds(k * L, L)] = jnp.bitwise_and(wz, 0xFFFF)
                dstb[b, pl.ds(k * L, L)] = jnp.right_shift(wz, 16)

        _unpack(0, 0, 0)
        pltpu.async_copy(x_hbm.at[srcb.at[0]], rows.at[0], sem0)

        def _step(j2, _):
            for b in range(2):
                nb = 1 - b
                nrow = j2 if b == 0 else j2 + 1
                noff = EC if b == 0 else 0

                @pl.when(2 * j2 + b + 1 < ch)
                def _():
                    _unpack(nrow, noff, nb)
                    pltpu.async_copy(
                        x_hbm.at[srcb.at[nb]], rows.at[nb], sems[nb]
                    )

                pltpu.make_async_copy(
                    x_hbm.at[srcb.at[b]], rows.at[b], sems[b]
                ).wait()
                pltpu.sync_copy(rows.at[b], acc.at[dstb.at[b]], add=True)
            return ()

        lax.fori_loop(0, ch // 2, _step, ())
        plsc.subcore_barrier()

        for z in range(zfull):
            pltpu.sync_copy(acc.at[pl.ds(r0 + z * 128, 128)], zb)
            pltpu.sync_copy(zb, out_hbm.at[c, pl.ds(r0 + z * 128, 128)])
        if zrem:
            pltpu.sync_copy(
                acc.at[pl.ds(r0 + zfull * 128, zrem)], zb.at[pl.ds(0, zrem)]
            )
            pltpu.sync_copy(
                zb.at[pl.ds(0, zrem)],
                out_hbm.at[c, pl.ds(r0 + zfull * 128, zrem)],
            )

    return _sc_agg


def _tc_body(p_ref, w_ref, b_ref, o_ref):
    o_ref[...] = (
        jnp.dot(
            p_ref[0] + p_ref[1], w_ref[...], preferred_element_type=jnp.float32
        )
        + b_ref[...]
    )


def _tc_combine(partials, Wp, bias, n_nodes: int):
    bm = 2000
    return pl.pallas_call(
        _tc_body,
        grid=(n_nodes // bm,),
        in_specs=[
            pl.BlockSpec((NC, bm, D), lambda i: (0, i, 0)),
            pl.BlockSpec((D, D), lambda i: (0, 0)),
            pl.BlockSpec((1, D), lambda i: (0, 0)),
        ],
        out_specs=pl.BlockSpec((bm, D), lambda i: (i, 0)),
        out_shape=jax.ShapeDtypeStruct((n_nodes, D), jnp.float32),
    )(partials, Wp, bias.reshape(1, D))


def kernel(x, edge_index, W, bias):
    n = x.shape[0]
    e = edge_index.shape[1]
    src = edge_index[0].astype(jnp.int32)
    dst = edge_index[1].astype(jnp.int32)

    block = NW * EC
    ch = (e + block - 1) // block
    ch += ch % 2
    e_pad = block * ch
    pad = e_pad - e
    src = jnp.concatenate([src, jnp.zeros((pad,), jnp.int32)])
    dst = jnp.concatenate([dst, jnp.full((pad,), n, jnp.int32)])
    packed = jnp.bitwise_or(src, jnp.left_shift(dst, 16))
    pk3 = packed.reshape(NW, ch // 2, 2 * EC)

    r_pad = ((n + 1 + 127) // 128) * 128
    partials = _make_sc_agg(n, ch, r_pad)(x, pk3)
    return _tc_combine(partials, W, bias, n)
